# baseline (speedup 1.0000x reference)
import jax
import jax.numpy as jnp
from jax import lax
from jax.experimental import pallas as pl
from jax.experimental.pallas import tpu as pltpu, tpu_sc as plsc

EPS = 1e-5
CHUNK = 128


def _message_pass_kernel(n_nodes, d, chunks_per_tile, num_subcores):
    W = 80
    n_wchunks = n_nodes // W
    max_wchunks_per_tile = -(-n_wchunks // num_subcores)
    ng = d // 16
    cpt = chunks_per_tile

    def body(table_hbm, p0_hbm, p1_hbm, attr_hbm,
             cagg_hbm, vagg_hbm,
             acc, b0, b1, b2, a0, a1, a2, r0, r1, r2,
             sg0, sg1, sg2, ss0, ss1, ss2, si0, si1, si2):
        bufs = [b0, b1, b2]
        avs = [a0, a1, a2]
        rows = [r0, r1, r2]
        sg = [sg0, sg1, sg2]
        ss = [ss0, ss1, ss2]
        si = [si0, si1, si2]
        c = lax.axis_index("c")
        s = lax.axis_index("s")
        g_base = s * cpt

        def idx_load(slot, g):
            @pl.when(c == 0)
            def _():
                pltpu.async_copy(p0_hbm.at[g], bufs[slot], si[slot])

            @pl.when(c == 1)
            def _():
                pltpu.async_copy(p1_hbm.at[g], bufs[slot], si[slot])
            pltpu.async_copy(attr_hbm.at[g], avs[slot], si[slot])

        def idx_wait(slot):
            pltpu.make_async_copy(p0_hbm.at[0], bufs[slot], si[slot]).wait()
            pltpu.make_async_copy(attr_hbm.at[0], avs[slot], si[slot]).wait()

        def gather_start(slot):
            pltpu.async_copy(table_hbm.at[bufs[slot].at[0]], rows[slot], sg[slot])

        def gather_wait(slot):
            pltpu.make_async_copy(
                table_hbm.at[bufs[slot].at[0]], rows[slot], sg[slot]).wait()

        def scatter_start(slot):
            pltpu.async_copy(rows[slot], acc.at[bufs[slot].at[1]], ss[slot],
                             add=True)

        def scatter_wait(slot):
            pltpu.make_async_copy(
                rows[slot], acc.at[bufs[slot].at[1]], ss[slot]).wait()

        def scale(slot):
            rws, av = rows[slot], avs[slot]

            def scale_group(gi, carry2):
                a16 = av[pl.ds(gi * 16, 16)]
                for j in range(16):
                    a = a16[j]
                    e = gi * 16 + j
                    for g in range(ng):
                        rws[e, pl.ds(g * 16, 16)] = rws[e, pl.ds(g * 16, 16)] * a
                return carry2
            lax.fori_loop(0, CHUNK // 16, scale_group, 0)

        def zero_row(r, carry):
            for g in range(ng):
                r0[r, pl.ds(g * 16, 16)] = jnp.zeros((16,), jnp.float32)
            return carry
        lax.fori_loop(0, CHUNK, zero_row, 0)

        def zero_chunk(i, carry):
            cid = s + i * num_subcores

            @pl.when(cid < n_wchunks)
            def _():
                rr = pl.multiple_of(cid * W, 8)
                pltpu.sync_copy(r0.at[pl.ds(0, W)], acc.at[pl.ds(rr, W)])
            return carry
        lax.fori_loop(0, max_wchunks_per_tile, zero_chunk, 0)
        plsc.subcore_barrier()

        def steady(k, u):
            slot, slot1, slot2 = u % 3, (u + 1) % 3, (u + 2) % 3
            gather_wait(slot)

            @pl.when(k + 1 < cpt)
            def _():
                idx_wait(slot1)
                gather_start(slot1)
            scale(slot)
            scatter_start(slot)

            @pl.when(k + 2 < cpt)
            def _():
                scatter_wait(slot2)
                idx_load(slot2, g_base + k + 2)

        idx_load(0, g_base)
        idx_load(1, g_base + 1)
        idx_wait(0)
        gather_start(0)
        gather_wait(0)
        idx_wait(1)
        gather_start(1)
        scale(0)
        scatter_start(0)
        idx_load(2, g_base + 2)
        gather_wait(1)
        idx_wait(2)
        gather_start(2)
        scale(1)
        scatter_start(1)
        scatter_wait(0)
        idx_load(0, g_base + 3)
        steady(2, 2)

        def tri(i, carry):
            k = 3 * i
            steady(k, 0)
            steady(k + 1, 1)
            steady(k + 2, 2)
            return carry
        lax.fori_loop(1, cpt // 3, tri, 0)

        scatter_wait(0)
        scatter_wait(1)
        scatter_wait(2)
        plsc.subcore_barrier()

        def wb_chunk(i, carry):
            cid = s + i * num_subcores

            @pl.when(cid < n_wchunks)
            def _():
                rr = pl.multiple_of(cid * W, 8)
                pltpu.sync_copy(acc.at[pl.ds(rr, W)], r0.at[pl.ds(0, W)])

                @pl.when(c == 0)
                def _():
                    pltpu.sync_copy(r0.at[pl.ds(0, W)], cagg_hbm.at[pl.ds(rr, W)])

                @pl.when(c == 1)
                def _():
                    pltpu.sync_copy(r0.at[pl.ds(0, W)], vagg_hbm.at[pl.ds(rr, W)])
            return carry
        lax.fori_loop(0, max_wchunks_per_tile, wb_chunk, 0)

    return body


def _message_pass(var_feats, constr_feats, src, dst, attr):
    n_nodes, d = var_feats.shape
    info = plsc.get_sparse_core_info()
    ns = info.num_subcores
    e_total = src.shape[0]
    n_chunks_tot = e_total // CHUNK
    chunks_per_tile = n_chunks_tot // ns

    table = jnp.concatenate([var_feats, constr_feats], axis=0)
    ar = attr.reshape(n_chunks_tot, CHUNK)
    sr = src.reshape(n_chunks_tot, CHUNK)
    dr = dst.reshape(n_chunks_tot, CHUNK)
    packed0 = jnp.stack([sr, dr], axis=1)
    packed1 = jnp.stack([dr + n_nodes, sr], axis=1)

    mesh = plsc.VectorSubcoreMesh(core_axis_name="c", subcore_axis_name="s")
    body = _message_pass_kernel(n_nodes, d, chunks_per_tile, ns)
    out_t = jax.ShapeDtypeStruct((n_nodes, d), jnp.float32)
    k = pl.kernel(
        body,
        out_type=(out_t, out_t),
        mesh=mesh,
        scratch_types=(
            [pltpu.VMEM_SHARED((n_nodes, d), jnp.float32)]
            + [pltpu.VMEM((2, CHUNK), jnp.int32)] * 3
            + [pltpu.VMEM((CHUNK,), jnp.float32)] * 3
            + [pltpu.VMEM((CHUNK, d), jnp.float32)] * 3
            + [pltpu.SemaphoreType.DMA] * 9
        ),
    )
    return k(table, packed0, packed1, ar)


def _mlp_body(x_ref, agg_ref, w1a_ref, w1b_ref, b1_ref, g_ref, bt_ref,
              w2_ref, b2_ref, out_ref):
    x = x_ref[...]
    h = jnp.dot(x, w1a_ref[...], preferred_element_type=jnp.float32)
    h = h + jnp.dot(agg_ref[...], w1b_ref[...], preferred_element_type=jnp.float32)
    h = h + b1_ref[...]
    mu = jnp.mean(h, axis=0, keepdims=True)
    var = jnp.mean((h - mu) ** 2, axis=0, keepdims=True)
    hn = (h - mu) * (g_ref[...] * lax.rsqrt(var + EPS)) + bt_ref[...]
    hr = jnp.maximum(hn, 0.0)
    out_ref[...] = x + jnp.dot(hr, w2_ref[...], preferred_element_type=jnp.float32) + b2_ref[...]


def _mlp_update(x, agg, W1, b1, g, bt, W2, b2):
    n, d = x.shape
    w1a = W1[:, :d].T
    w1b = W1[:, d:].T
    return pl.pallas_call(
        _mlp_body,
        out_shape=jax.ShapeDtypeStruct((n, d), jnp.float32),
    )(x, agg, w1a, w1b, b1.reshape(1, -1), g.reshape(1, -1),
      bt.reshape(1, -1), W2.T, b2.reshape(1, -1))


def kernel(var_feats, constr_feats, edge_index, edge_attr,
           W1, b1, g1, bt1, W2, b2, W3, b3, g2, bt2, W4, b4):
    n_edges = edge_index.shape[1]
    info = plsc.get_sparse_core_info()
    ns = info.num_subcores
    cpt = -(-n_edges // (ns * 3 * CHUNK)) * 3
    e_pad = cpt * CHUNK * ns
    pad = e_pad - n_edges
    src = jnp.pad(edge_index[0], (0, pad))
    dst = jnp.pad(edge_index[1], (0, pad))
    attr = jnp.pad(edge_attr, (0, pad))

    constr_agg, var_agg = _message_pass(var_feats, constr_feats, src, dst, attr)
    var_updated = _mlp_update(var_feats, var_agg, W1, b1, g1, bt1, W2, b2)
    constr_updated = _mlp_update(constr_feats, constr_agg, W3, b3, g2, bt2, W4, b4)
    return (var_updated, constr_updated)


# speedup vs baseline: 1.2513x; 1.2513x over previous
import jax
import jax.numpy as jnp
from jax import lax
from jax.experimental import pallas as pl
from jax.experimental.pallas import tpu as pltpu, tpu_sc as plsc

EPS = 1e-5
CHUNK = 128


def _message_pass_kernel(n_nodes, d, chunks_per_tile, num_subcores):
    W = 80
    n_wchunks = n_nodes // W
    max_wchunks_per_tile = -(-n_wchunks // num_subcores)
    ng = d // 16

    def body(table_hbm, p0_hbm, p1_hbm, attr_hbm,
             cagg_hbm, vagg_hbm,
             acc, buf0, buf1, av0, av1, rows0, rows1,
             sem0, sem1, si0, si1):
        c = lax.axis_index("c")
        s = lax.axis_index("s")

        def load_buf(buf, av, si, g):
            @pl.when(c == 0)
            def _():
                pltpu.async_copy(p0_hbm.at[g], buf, si)

            @pl.when(c == 1)
            def _():
                pltpu.async_copy(p1_hbm.at[g], buf, si)
            pltpu.async_copy(attr_hbm.at[g], av, si)
            pltpu.make_async_copy(p0_hbm.at[0], buf, si).wait()
            pltpu.make_async_copy(attr_hbm.at[0], av, si).wait()

        def start_gather(buf, rows, sem):
            pltpu.async_copy(table_hbm.at[buf.at[0]], rows, sem)

        def finish_chunk(buf, av, rows, sem):
            pltpu.make_async_copy(table_hbm.at[buf.at[0]], rows, sem).wait()

            def scale_group(gi, carry2):
                a16 = av[pl.ds(gi * 16, 16)]
                for j in range(16):
                    a = a16[j]
                    e = gi * 16 + j
                    for g in range(ng):
                        rows[e, pl.ds(g * 16, 16)] = rows[e, pl.ds(g * 16, 16)] * a
                return carry2
            lax.fori_loop(0, CHUNK // 16, scale_group, 0, unroll=2)
            pltpu.sync_copy(rows, acc.at[buf.at[1]], add=True)

        def zero_row(r, carry):
            for g in range(ng):
                rows0[r, pl.ds(g * 16, 16)] = jnp.zeros((16,), jnp.float32)
            return carry
        lax.fori_loop(0, CHUNK, zero_row, 0)

        def zero_chunk(i, carry):
            cid = s + i * num_subcores

            @pl.when(cid < n_wchunks)
            def _():
                r0 = pl.multiple_of(cid * W, 8)
                pltpu.sync_copy(rows0.at[pl.ds(0, W)], acc.at[pl.ds(r0, W)])
            return carry
        lax.fori_loop(0, max_wchunks_per_tile, zero_chunk, 0)
        plsc.subcore_barrier()

        g_base = s * chunks_per_tile
        load_buf(buf0, av0, si0, g_base)
        start_gather(buf0, rows0, sem0)

        def pair(i, carry):
            g = g_base + 2 * i
            load_buf(buf1, av1, si1, g + 1)
            start_gather(buf1, rows1, sem1)
            finish_chunk(buf0, av0, rows0, sem0)

            @pl.when(2 * i + 2 < chunks_per_tile)
            def _():
                load_buf(buf0, av0, si0, g + 2)
                start_gather(buf0, rows0, sem0)
            finish_chunk(buf1, av1, rows1, sem1)
            return carry
        lax.fori_loop(0, chunks_per_tile // 2, pair, 0)
        plsc.subcore_barrier()

        def wb_chunk(i, carry):
            cid = s + i * num_subcores

            @pl.when(cid < n_wchunks)
            def _():
                r0 = pl.multiple_of(cid * W, 8)
                pltpu.sync_copy(acc.at[pl.ds(r0, W)], rows0.at[pl.ds(0, W)])

                @pl.when(c == 0)
                def _():
                    pltpu.sync_copy(rows0.at[pl.ds(0, W)], cagg_hbm.at[pl.ds(r0, W)])

                @pl.when(c == 1)
                def _():
                    pltpu.sync_copy(rows0.at[pl.ds(0, W)], vagg_hbm.at[pl.ds(r0, W)])
            return carry
        lax.fori_loop(0, max_wchunks_per_tile, wb_chunk, 0)

    return body


def _message_pass(var_feats, constr_feats, src, dst, attr):
    n_nodes, d = var_feats.shape
    info = plsc.get_sparse_core_info()
    ns = info.num_subcores
    e_total = src.shape[0]
    n_chunks_tot = e_total // CHUNK
    chunks_per_tile = n_chunks_tot // ns

    table = jnp.concatenate([var_feats, constr_feats], axis=0)
    ar = attr.reshape(n_chunks_tot, CHUNK)
    sr = src.reshape(n_chunks_tot, CHUNK)
    dr = dst.reshape(n_chunks_tot, CHUNK)
    packed0 = jnp.stack([sr, dr], axis=1)
    packed1 = jnp.stack([dr + n_nodes, sr], axis=1)

    mesh = plsc.VectorSubcoreMesh(core_axis_name="c", subcore_axis_name="s")
    body = _message_pass_kernel(n_nodes, d, chunks_per_tile, ns)
    out_t = jax.ShapeDtypeStruct((n_nodes, d), jnp.float32)
    k = pl.kernel(
        body,
        out_type=(out_t, out_t),
        mesh=mesh,
        scratch_types=[
            pltpu.VMEM_SHARED((n_nodes, d), jnp.float32),
            pltpu.VMEM((2, CHUNK), jnp.int32),
            pltpu.VMEM((2, CHUNK), jnp.int32),
            pltpu.VMEM((CHUNK,), jnp.float32),
            pltpu.VMEM((CHUNK,), jnp.float32),
            pltpu.VMEM((CHUNK, d), jnp.float32),
            pltpu.VMEM((CHUNK, d), jnp.float32),
            pltpu.SemaphoreType.DMA,
            pltpu.SemaphoreType.DMA,
            pltpu.SemaphoreType.DMA,
            pltpu.SemaphoreType.DMA,
        ],
    )
    return k(table, packed0, packed1, ar)


def _mlp_body(x_ref, agg_ref, w1a_ref, w1b_ref, b1_ref, g_ref, bt_ref,
              w2_ref, b2_ref, out_ref):
    x = x_ref[...]
    h = jnp.dot(x, w1a_ref[...], preferred_element_type=jnp.float32)
    h = h + jnp.dot(agg_ref[...], w1b_ref[...], preferred_element_type=jnp.float32)
    h = h + b1_ref[...]
    mu = jnp.mean(h, axis=0, keepdims=True)
    var = jnp.mean((h - mu) ** 2, axis=0, keepdims=True)
    hn = (h - mu) * (g_ref[...] * lax.rsqrt(var + EPS)) + bt_ref[...]
    hr = jnp.maximum(hn, 0.0)
    out_ref[...] = x + jnp.dot(hr, w2_ref[...], preferred_element_type=jnp.float32) + b2_ref[...]


def _mlp_update(x, agg, W1, b1, g, bt, W2, b2):
    n, d = x.shape
    w1a = W1[:, :d].T
    w1b = W1[:, d:].T
    return pl.pallas_call(
        _mlp_body,
        out_shape=jax.ShapeDtypeStruct((n, d), jnp.float32),
    )(x, agg, w1a, w1b, b1.reshape(1, -1), g.reshape(1, -1),
      bt.reshape(1, -1), W2.T, b2.reshape(1, -1))


def kernel(var_feats, constr_feats, edge_index, edge_attr,
           W1, b1, g1, bt1, W2, b2, W3, b3, g2, bt2, W4, b4):
    n_edges = edge_index.shape[1]
    info = plsc.get_sparse_core_info()
    ns = info.num_subcores
    per_tile = -(-n_edges // (ns * 2 * CHUNK)) * 2 * CHUNK
    e_pad = per_tile * ns
    pad = e_pad - n_edges
    src = jnp.pad(edge_index[0], (0, pad))
    dst = jnp.pad(edge_index[1], (0, pad))
    attr = jnp.pad(edge_attr, (0, pad))

    constr_agg, var_agg = _message_pass(var_feats, constr_feats, src, dst, attr)
    var_updated = _mlp_update(var_feats, var_agg, W1, b1, g1, bt1, W2, b2)
    constr_updated = _mlp_update(constr_feats, constr_agg, W3, b3, g2, bt2, W4, b4)
    return (var_updated, constr_updated)
